# Initial kernel scaffold; baseline (speedup 1.0000x reference)
import jax
import jax.numpy as jnp
from jax.experimental import pallas as pl


def kernel(src_pos, table):
    raise NotImplementedError("write your pallas kernel here")


# speedup vs baseline: 5.0994x; 5.0994x over previous
import functools
import jax
import jax.numpy as jnp
from jax import lax
from jax.experimental import pallas as pl
from jax.experimental.pallas import tpu as pltpu
from jax.experimental.pallas import tpu_sc as plsc

NC = 2
NS = 16
NW = NC * NS
GROUP = 128
BLK = 4


@functools.partial(jax.jit, static_argnums=(2, 3))
def _gather(table, idx2d, n_rows, d):
  rows_per_w = n_rows // NW
  nb = rows_per_w // BLK
  assert rows_per_w % BLK == 0 and nb % 2 == 0 and nb >= 4

  mesh = plsc.VectorSubcoreMesh(core_axis_name="c", subcore_axis_name="s")

  @functools.partial(
      pl.kernel,
      out_type=jax.ShapeDtypeStruct((n_rows, GROUP, d), jnp.float32),
      mesh=mesh,
      compiler_params=pltpu.CompilerParams(use_tc_tiling_on_sc=False),
      scratch_types=[
          pltpu.VMEM((2, BLK, GROUP), jnp.int32),
          pltpu.VMEM((2, BLK, GROUP, d), jnp.float32),
          pltpu.SemaphoreType.DMA,
          pltpu.SemaphoreType.DMA,
          pltpu.SemaphoreType.DMA,
          pltpu.SemaphoreType.DMA,
          pltpu.SemaphoreType.DMA,
          pltpu.SemaphoreType.DMA,
      ],
  )
  def k(table_hbm, idx_hbm, out_hbm, idx_v, rows_v,
        isem0, isem1, gsem0, gsem1, ssem0, ssem1):
    isem = (isem0, isem1)
    gsem = (gsem0, gsem1)
    ssem = (ssem0, ssem1)
    wid = lax.axis_index("s") * NC + lax.axis_index("c")
    base = wid * rows_per_w

    def idx_copy(b, p):
      return pltpu.make_async_copy(
          idx_hbm.at[pl.ds(base + b * BLK, BLK)], idx_v.at[p], isem[p])

    def gather_copies(b, p):
      del b
      return [
          pltpu.make_async_copy(
              table_hbm.at[idx_v.at[p, j]], rows_v.at[p, j], gsem[p])
          for j in range(BLK)
      ]

    def store_copy(b, p):
      return pltpu.make_async_copy(
          rows_v.at[p], out_hbm.at[pl.ds(base + b * BLK, BLK)], ssem[p])

    def block_iter(b, p, idx_next2, gather_next, wait_store_next):
      q = 1 - p
      for c in gather_copies(b, p):
        c.wait()
      store_copy(b, p).start()
      if idx_next2:
        idx_copy(b + 2, p).start()
      if gather_next:
        idx_copy(b + 1, q).wait()
        if wait_store_next:
          store_copy(b - 1, q).wait()
        for c in gather_copies(b + 1, q):
          c.start()

    idx_copy(0, 0).start()
    idx_copy(1, 1).start()
    idx_copy(0, 0).wait()
    for c in gather_copies(0, 0):
      c.start()

    block_iter(0, 0, True, True, False)
    block_iter(1, 1, True, True, True)

    @pl.loop(1, nb // 2 - 1)
    def _(i):
      b = i * 2
      block_iter(b, 0, True, True, True)
      block_iter(b + 1, 1, True, True, True)

    block_iter(nb - 2, 0, False, True, True)
    block_iter(nb - 1, 1, False, False, False)

    store_copy(nb - 2, 0).wait()
    store_copy(nb - 1, 1).wait()

  return k(table, idx2d)


def kernel(src_pos, table):
  b, h = src_pos.shape
  d = table.shape[1]
  n_rows = (b * h) // GROUP
  idx2d = src_pos.reshape(n_rows, GROUP)
  out = _gather(table, idx2d, n_rows, d)
  return out.reshape(b, h, d)


# speedup vs baseline: 5.8015x; 1.1377x over previous
import functools
import jax
import jax.numpy as jnp
from jax import lax
from jax.experimental import pallas as pl
from jax.experimental.pallas import tpu as pltpu
from jax.experimental.pallas import tpu_sc as plsc

NC = 2
NS = 16
NW = NC * NS
GROUP = 128
BLK = 4


@functools.partial(jax.jit, static_argnums=(2, 3))
def _gather(table, idx2d, n_rows, d):
  n_table = table.shape[0]
  rows_per_w = n_rows // NW
  nb = rows_per_w // BLK
  assert rows_per_w % BLK == 0 and nb % 2 == 0 and nb >= 4

  mesh = plsc.VectorSubcoreMesh(core_axis_name="c", subcore_axis_name="s")

  @functools.partial(
      pl.kernel,
      out_type=jax.ShapeDtypeStruct((n_rows, GROUP, d), jnp.float32),
      mesh=mesh,
      compiler_params=pltpu.CompilerParams(use_tc_tiling_on_sc=False),
      scratch_types=[
          pltpu.VMEM((2, BLK, GROUP), jnp.int32),
          pltpu.VMEM((2, BLK, GROUP, d), jnp.float32),
          pltpu.VMEM_SHARED((n_table, d), jnp.float32),
          pltpu.SemaphoreType.DMA,
          pltpu.SemaphoreType.DMA,
          pltpu.SemaphoreType.DMA,
          pltpu.SemaphoreType.DMA,
          pltpu.SemaphoreType.DMA,
          pltpu.SemaphoreType.DMA,
      ],
  )
  def k(table_hbm, idx_hbm, out_hbm, idx_v, rows_v, table_sh,
        isem0, isem1, gsem0, gsem1, ssem0, ssem1):
    isem = (isem0, isem1)
    gsem = (gsem0, gsem1)
    ssem = (ssem0, ssem1)
    wid = lax.axis_index("s") * NC + lax.axis_index("c")
    base = wid * rows_per_w

    @pl.when(lax.axis_index("s") == 0)
    def _():
      pltpu.sync_copy(table_hbm, table_sh)

    plsc.subcore_barrier()

    def idx_copy(b, p):
      return pltpu.make_async_copy(
          idx_hbm.at[pl.ds(base + b * BLK, BLK)], idx_v.at[p], isem[p])

    def gather_copies(b, p):
      del b
      return [
          pltpu.make_async_copy(
              table_sh.at[idx_v.at[p, j]], rows_v.at[p, j], gsem[p])
          for j in range(BLK)
      ]

    def store_copy(b, p):
      return pltpu.make_async_copy(
          rows_v.at[p], out_hbm.at[pl.ds(base + b * BLK, BLK)], ssem[p])

    def block_iter(b, p, idx_next2, gather_next, wait_store_next):
      q = 1 - p
      for c in gather_copies(b, p):
        c.wait()
      store_copy(b, p).start()
      if idx_next2:
        idx_copy(b + 2, p).start()
      if gather_next:
        idx_copy(b + 1, q).wait()
        if wait_store_next:
          store_copy(b - 1, q).wait()
        for c in gather_copies(b + 1, q):
          c.start()

    idx_copy(0, 0).start()
    idx_copy(1, 1).start()
    idx_copy(0, 0).wait()
    for c in gather_copies(0, 0):
      c.start()

    block_iter(0, 0, True, True, False)
    block_iter(1, 1, True, True, True)

    @pl.loop(1, nb // 2 - 1)
    def _(i):
      b = i * 2
      block_iter(b, 0, True, True, True)
      block_iter(b + 1, 1, True, True, True)

    block_iter(nb - 2, 0, False, True, True)
    block_iter(nb - 1, 1, False, False, False)

    store_copy(nb - 2, 0).wait()
    store_copy(nb - 1, 1).wait()

  return k(table, idx2d)


def kernel(src_pos, table):
  b, h = src_pos.shape
  d = table.shape[1]
  n_rows = (b * h) // GROUP
  idx2d = src_pos.reshape(n_rows, GROUP)
  out = _gather(table, idx2d, n_rows, d)
  return out.reshape(b, h, d)
